# baseline (speedup 1.0000x reference)
import functools

import jax
import jax.numpy as jnp
from jax import lax
from jax.experimental import pallas as pl
from jax.experimental.pallas import tpu as pltpu
from jax.experimental.pallas import tpu_sc as plsc

NC = 2
NS = 16
NW = NC * NS
L = 16
K = 128


def _mesh():
    return plsc.VectorSubcoreMesh(
        core_axis_name="c", subcore_axis_name="s", num_cores=NC,
        num_subcores=NS)


def _norm_body(chunks, npad, row_h, col_h, ew_h, norm_h,
               row_v, col_v, ew_v, dis_v, nrm_v, t_v, acc_sp, sem_s):
    cid = lax.axis_index("c")
    sid = lax.axis_index("s")
    npt = npad // NS
    cpt = 2 * chunks

    @pl.when(cid == 0)
    def _():
        zero = jnp.zeros((L,), jnp.float32)
        pltpu.sync_copy(col_h.at[pl.ds(cpt * sid, cpt)], col_v)
        pltpu.sync_copy(ew_h.at[pl.ds(cpt * sid, cpt)], ew_v)
        for i in range(npt // L):
            t_v[pl.ds(L * i, L)] = zero
        pltpu.sync_copy(t_v, acc_sp.at[pl.ds(npt * sid, npt)])
        plsc.subcore_barrier()

        def step_a(ci, carry):
            pltpu.async_copy(ew_v.at[ci], acc_sp.at[col_v.at[ci]], sem_s,
                             add=True)
            return carry
        lax.fori_loop(0, cpt, step_a, 0)

        def drain_a(ci, carry):
            pltpu.make_async_copy(ew_v.at[0], acc_sp.at[col_v.at[0]],
                                  sem_s).wait()
            return carry
        lax.fori_loop(0, cpt, drain_a, 0)
        plsc.subcore_barrier()

        pltpu.sync_copy(acc_sp.at[pl.ds(npt * sid, npt)], t_v)
        for i in range(npt // L):
            d = t_v[pl.ds(L * i, L)]
            yi = lax.bitcast_convert_type(d, jnp.int32)
            yi = jnp.int32(0x5F3759DF) - lax.shift_right_logical(yi, 1)
            y = lax.bitcast_convert_type(yi, jnp.float32)
            for _ in range(3):
                y = y * (1.5 - 0.5 * d * y * y)
            t_v[pl.ds(L * i, L)] = y
        pltpu.sync_copy(t_v, acc_sp.at[pl.ds(npt * sid, npt)])
        plsc.subcore_barrier()
        pltpu.sync_copy(acc_sp, dis_v)

        pltpu.sync_copy(row_h.at[pl.ds(cpt * sid, cpt)], row_v)
        lane = lax.iota(jnp.int32, L)

        @plsc.parallel_loop(0, cpt * (K // L), 1, unroll=4)
        def _(g):
            ci16 = jnp.full((L,), g // (K // L), jnp.int32)
            off = lane + L * (g % (K // L))
            r16 = plsc.load_gather(row_v, [ci16, off])
            c16 = plsc.load_gather(col_v, [ci16, off])
            w16 = plsc.load_gather(ew_v, [ci16, off])
            dr = plsc.load_gather(dis_v, [r16])
            dc = plsc.load_gather(dis_v, [c16])
            plsc.store_scatter(nrm_v, [ci16, off], dr * w16 * dc)
        pltpu.sync_copy(nrm_v, norm_h.at[pl.ds(cpt * sid, cpt)])


def _make_norm_kernel(chunks, npad):
    cpt = 2 * chunks
    return pl.kernel(
        functools.partial(_norm_body, chunks, npad),
        out_type=jax.ShapeDtypeStruct((NW * chunks, K), jnp.float32),
        mesh=_mesh(),
        compiler_params=pltpu.CompilerParams(needs_layout_passes=False),
        scratch_types=[
            pltpu.VMEM((cpt, K), jnp.int32),
            pltpu.VMEM((cpt, K), jnp.int32),
            pltpu.VMEM((cpt, K), jnp.float32),
            pltpu.VMEM((npad,), jnp.float32),
            pltpu.VMEM((cpt, K), jnp.float32),
            pltpu.VMEM((npad // NS,), jnp.float32),
            pltpu.VMEM_SHARED((npad,), jnp.float32),
            pltpu.SemaphoreType.DMA,
        ],
    )


def _msg_body(chunks, npad, d,
              h_h, row_h, col_h, nrm_h, a0_h, a1_h,
              rowb, colb, nrmb, rows_a, rows_b, stage, acc_sp,
              sem_a, sem_b, sem_p):
    cid = lax.axis_index("c")
    sid = lax.axis_index("s")
    wid = 2 * sid + cid
    npt = npad // NS
    nv = d // L
    base = chunks * wid
    nblk = chunks // 8

    zero = jnp.zeros((L,), jnp.float32)

    def zr(r, carry):
        for v in range(nv):
            stage[r, pl.ds(L * v, L)] = zero
        return carry
    lax.fori_loop(0, K, zr, 0)
    for k in range(npt // K):
        pltpu.sync_copy(stage, acc_sp.at[pl.ds(npt * sid + K * k, K)])
    pltpu.sync_copy(row_h.at[pl.ds(base, 8)], rowb.at[0])
    pltpu.sync_copy(col_h.at[pl.ds(base, 8)], colb.at[0])
    pltpu.sync_copy(nrm_h.at[pl.ds(base, 8)], nrmb.at[0])
    plsc.subcore_barrier()
    pltpu.async_copy(h_h.at[rowb.at[0].at[0]], rows_a, sem_a)
    pltpu.async_copy(h_h.at[rowb.at[0].at[1]], rows_b, sem_b)

    lane = lax.iota(jnp.int32, L)
    lane2 = lane * 2

    def scale(rows_v, par, j):
        j16 = jnp.full((L,), j, jnp.int32)
        p16 = jnp.full((L,), par, jnp.int32)

        @plsc.parallel_loop(0, K, 1, unroll=4)
        def _(e):
            e16 = jnp.full((L,), e, jnp.int32)
            s16 = plsc.load_gather(nrmb, [p16, j16, e16])
            sbf = plsc.pack(s16, s16, format=plsc.PackFormat.INTERLEAVED)
            for q in range(nv // 2):
                w = plsc.load_gather(rows_v, [e16, lane + L * q])
                hv = plsc.bitcast(w, jnp.bfloat16)
                prod = hv * sbf
                pa, pb = plsc.unpack(prod,
                                     format=plsc.PackFormat.INTERLEAVED)
                plsc.store_scatter(stage, [e16, lane2 + 2 * L * q], pa)
                plsc.store_scatter(stage, [e16, lane2 + 2 * L * q + 1], pb)

    def halfstep(rows_v, sem, c, par, j):
        pltpu.make_async_copy(h_h.at[pl.ds(0, K)], rows_v, sem).wait()
        scale(rows_v, par, j)
        pltpu.sync_copy(stage, acc_sp.at[colb.at[par].at[j]], add=True)
        c2 = c + 2

        @pl.when(c2 < chunks)
        def _():
            blk2 = c2 // 8
            pltpu.async_copy(
                h_h.at[rowb.at[blk2 % 2].at[c2 - 8 * blk2]], rows_v, sem)

    def step(i, carry):
        c0 = 2 * i
        blk = i // 4
        par = blk % 2
        j0 = c0 - 8 * blk

        @pl.when((j0 == 0) & (blk + 1 < nblk))
        def _():
            src = base + 8 * (blk + 1)
            pp = 1 - par
            pltpu.async_copy(row_h.at[pl.ds(src, 8)], rowb.at[pp], sem_p)
            pltpu.async_copy(col_h.at[pl.ds(src, 8)], colb.at[pp], sem_p)
            pltpu.async_copy(nrm_h.at[pl.ds(src, 8)], nrmb.at[pp], sem_p)

        @pl.when((j0 == 6) & (blk + 1 < nblk))
        def _():
            pp = 1 - par
            pltpu.make_async_copy(row_h.at[pl.ds(0, 8)],
                                  rowb.at[pp], sem_p).wait()
            pltpu.make_async_copy(col_h.at[pl.ds(0, 8)],
                                  colb.at[pp], sem_p).wait()
            pltpu.make_async_copy(nrm_h.at[pl.ds(0, 8)],
                                  nrmb.at[pp], sem_p).wait()

        halfstep(rows_a, sem_a, c0, par, j0)
        halfstep(rows_b, sem_b, c0 + 1, par, j0 + 1)
        return carry
    lax.fori_loop(0, chunks // 2, step, 0)
    plsc.subcore_barrier()

    @pl.when(cid == 0)
    def _():
        pltpu.sync_copy(acc_sp.at[pl.ds(npt * sid, npt)],
                        a0_h.at[pl.ds(npt * sid, npt)])

    @pl.when(cid == 1)
    def _():
        pltpu.sync_copy(acc_sp.at[pl.ds(npt * sid, npt)],
                        a1_h.at[pl.ds(npt * sid, npt)])


def _make_msg_kernel(chunks, npad, d):
    return pl.kernel(
        functools.partial(_msg_body, chunks, npad, d),
        out_type=[jax.ShapeDtypeStruct((npad, d), jnp.float32),
                  jax.ShapeDtypeStruct((npad, d), jnp.float32)],
        mesh=_mesh(),
        compiler_params=pltpu.CompilerParams(needs_layout_passes=False,
                                             use_tc_tiling_on_sc=False),
        scratch_types=[
            pltpu.VMEM((2, 8, K), jnp.int32),
            pltpu.VMEM((2, 8, K), jnp.int32),
            pltpu.VMEM((2, 8, K), jnp.float32),
            pltpu.VMEM((K, d // 2), jnp.int32),
            pltpu.VMEM((K, d // 2), jnp.int32),
            pltpu.VMEM((K, d), jnp.float32),
            pltpu.VMEM_SHARED((npad, d), jnp.float32),
            pltpu.SemaphoreType.DMA,
            pltpu.SemaphoreType.DMA,
            pltpu.SemaphoreType.DMA,
        ],
    )


def _pick_bm(n):
    for bm in (1000, 500, 200, 100, 8):
        if n % bm == 0:
            return bm
    return n


def _tc_matmul(x, w, out_dtype=jnp.float32):
    n, d = x.shape
    bm = _pick_bm(n)

    def body(x_ref, w_ref, o_ref):
        o_ref[...] = jnp.dot(x_ref[...], w_ref[...],
                             preferred_element_type=jnp.float32
                             ).astype(out_dtype)
    return pl.pallas_call(
        body,
        grid=(n // bm,),
        in_specs=[pl.BlockSpec((bm, d), lambda j: (j, 0)),
                  pl.BlockSpec(w.shape, lambda j: (0, 0))],
        out_specs=pl.BlockSpec((bm, w.shape[1]), lambda j: (j, 0)),
        out_shape=jax.ShapeDtypeStruct((n, w.shape[1]), out_dtype),
    )(x, w)


def _tc_relu_matmul(n, a0, a1, b, w, bout=None, out_dtype=jnp.float32):
    d = a0.shape[1]
    bm = _pick_bm(n)
    b2 = b.reshape(1, d)
    args = [a0, a1, b2, w]
    in_specs = [pl.BlockSpec((bm, d), lambda j: (j, 0)),
                pl.BlockSpec((bm, d), lambda j: (j, 0)),
                pl.BlockSpec((1, d), lambda j: (0, 0)),
                pl.BlockSpec(w.shape, lambda j: (0, 0))]
    if bout is not None:
        args.append(bout.reshape(1, w.shape[1]))
        in_specs.append(pl.BlockSpec((1, w.shape[1]), lambda j: (0, 0)))

    def body(a0_ref, a1_ref, b_ref, w_ref, *rest):
        o_ref = rest[-1]
        t = jnp.maximum(a0_ref[...] + a1_ref[...] + b_ref[...], 0.0)
        o = jnp.dot(t, w_ref[...], preferred_element_type=jnp.float32)
        if bout is not None:
            o = o + rest[0][...]
        o_ref[...] = o.astype(out_dtype)
    return pl.pallas_call(
        body,
        grid=(n // bm,),
        in_specs=in_specs,
        out_specs=pl.BlockSpec((bm, w.shape[1]), lambda j: (j, 0)),
        out_shape=jax.ShapeDtypeStruct((n, w.shape[1]), out_dtype),
    )(*args)


def kernel(x, edge_index, edge_attr, W1, b1, W2, b2, Wout, bout):
    n, d = x.shape
    e = edge_index.shape[1]

    loop = jnp.arange(n, dtype=jnp.int32)
    ep_real = e + n
    chunks = 8 * (-(-ep_real // (NW * K * 8)))
    ep = NW * K * chunks
    pad = ep - ep_real
    padi = jnp.arange(pad, dtype=jnp.int32)
    row = jnp.concatenate(
        [edge_index[0].astype(jnp.int32), loop,
         padi % n]).reshape(NW * chunks, K)
    col = jnp.concatenate(
        [edge_index[1].astype(jnp.int32), loop,
         padi % n]).reshape(NW * chunks, K)
    ew = jnp.concatenate(
        [edge_attr.astype(jnp.float32), jnp.ones((n,), jnp.float32),
         jnp.zeros((pad,), jnp.float32)]).reshape(NW * chunks, K)

    npt = -(-n // (NS * K)) * K
    npad = NS * npt

    norm = _make_norm_kernel(chunks, npad)(row, col, ew)
    msg = _make_msg_kernel(chunks, npad, d)

    def as_i32_rows(h_bf16):
        return lax.bitcast_convert_type(
            h_bf16.reshape(n, d // 2, 2), jnp.int32)

    h1 = _tc_matmul(x.astype(jnp.float32), W1, out_dtype=jnp.bfloat16)
    a0, a1 = msg(as_i32_rows(h1), row, col, norm)
    h2 = _tc_relu_matmul(n, a0, a1, b1, W2, out_dtype=jnp.bfloat16)
    c0, c1 = msg(as_i32_rows(h2), row, col, norm)
    out = _tc_relu_matmul(n, c0, c1, b2, Wout, bout)
    return out


# speedup vs baseline: 1.0266x; 1.0266x over previous
import functools

import jax
import jax.numpy as jnp
from jax import lax
from jax.experimental import pallas as pl
from jax.experimental.pallas import tpu as pltpu
from jax.experimental.pallas import tpu_sc as plsc

NC = 2
NS = 16
NW = NC * NS
L = 16
K = 128


def _mesh():
    return plsc.VectorSubcoreMesh(
        core_axis_name="c", subcore_axis_name="s", num_cores=NC,
        num_subcores=NS)


def _norm_body(chunks, npad, row_h, col_h, ew_h, norm_h,
               row_v, col_v, ew_v, dis_v, nrm_v, t_v, acc_sp, sem_s):
    cid = lax.axis_index("c")
    sid = lax.axis_index("s")
    npt = npad // NS
    cpt = 2 * chunks

    @pl.when(cid == 0)
    def _():
        zero = jnp.zeros((L,), jnp.float32)
        pltpu.sync_copy(col_h.at[pl.ds(cpt * sid, cpt)], col_v)
        pltpu.sync_copy(ew_h.at[pl.ds(cpt * sid, cpt)], ew_v)
        for i in range(npt // L):
            t_v[pl.ds(L * i, L)] = zero
        pltpu.sync_copy(t_v, acc_sp.at[pl.ds(npt * sid, npt)])
        plsc.subcore_barrier()

        def step_a(ci, carry):
            pltpu.async_copy(ew_v.at[ci], acc_sp.at[col_v.at[ci]], sem_s,
                             add=True)
            return carry
        lax.fori_loop(0, cpt, step_a, 0)

        def drain_a(ci, carry):
            pltpu.make_async_copy(ew_v.at[0], acc_sp.at[col_v.at[0]],
                                  sem_s).wait()
            return carry
        lax.fori_loop(0, cpt, drain_a, 0)
        plsc.subcore_barrier()

        pltpu.sync_copy(acc_sp.at[pl.ds(npt * sid, npt)], t_v)
        for i in range(npt // L):
            d = t_v[pl.ds(L * i, L)]
            yi = lax.bitcast_convert_type(d, jnp.int32)
            yi = jnp.int32(0x5F3759DF) - lax.shift_right_logical(yi, 1)
            y = lax.bitcast_convert_type(yi, jnp.float32)
            for _ in range(3):
                y = y * (1.5 - 0.5 * d * y * y)
            t_v[pl.ds(L * i, L)] = y
        pltpu.sync_copy(t_v, acc_sp.at[pl.ds(npt * sid, npt)])
        plsc.subcore_barrier()
        pltpu.sync_copy(acc_sp, dis_v)

        pltpu.sync_copy(row_h.at[pl.ds(cpt * sid, cpt)], row_v)
        lane = lax.iota(jnp.int32, L)

        @plsc.parallel_loop(0, cpt * (K // L), 1, unroll=4)
        def _(g):
            ci16 = jnp.full((L,), g // (K // L), jnp.int32)
            off = lane + L * (g % (K // L))
            r16 = plsc.load_gather(row_v, [ci16, off])
            c16 = plsc.load_gather(col_v, [ci16, off])
            w16 = plsc.load_gather(ew_v, [ci16, off])
            dr = plsc.load_gather(dis_v, [r16])
            dc = plsc.load_gather(dis_v, [c16])
            plsc.store_scatter(nrm_v, [ci16, off], dr * w16 * dc)
        pltpu.sync_copy(nrm_v, norm_h.at[pl.ds(cpt * sid, cpt)])


def _make_norm_kernel(chunks, npad):
    cpt = 2 * chunks
    return pl.kernel(
        functools.partial(_norm_body, chunks, npad),
        out_type=jax.ShapeDtypeStruct((NW * chunks, K), jnp.float32),
        mesh=_mesh(),
        compiler_params=pltpu.CompilerParams(needs_layout_passes=False),
        scratch_types=[
            pltpu.VMEM((cpt, K), jnp.int32),
            pltpu.VMEM((cpt, K), jnp.int32),
            pltpu.VMEM((cpt, K), jnp.float32),
            pltpu.VMEM((npad,), jnp.float32),
            pltpu.VMEM((cpt, K), jnp.float32),
            pltpu.VMEM((npad // NS,), jnp.float32),
            pltpu.VMEM_SHARED((npad,), jnp.float32),
            pltpu.SemaphoreType.DMA,
        ],
    )


def _msg_body(chunks, npad, d,
              h_h, row_h, col_h, nrm_h, a0_h, a1_h,
              rowb, colb, nrmb, rows_a, rows_b, stage, acc_sp,
              sem_a, sem_b, sem_p):
    cid = lax.axis_index("c")
    sid = lax.axis_index("s")
    wid = 2 * sid + cid
    npt = npad // NS
    nv = d // L
    base = chunks * wid
    nblk = chunks // 8

    zero = jnp.zeros((L,), jnp.float32)

    def zr(r, carry):
        for v in range(nv):
            stage[r, pl.ds(L * v, L)] = zero
        return carry
    lax.fori_loop(0, K, zr, 0)
    for k in range(npt // K):
        pltpu.sync_copy(stage, acc_sp.at[pl.ds(npt * sid + K * k, K)])
    pltpu.sync_copy(row_h.at[pl.ds(base, 8)], rowb.at[0])
    pltpu.sync_copy(col_h.at[pl.ds(base, 8)], colb.at[0])
    pltpu.sync_copy(nrm_h.at[pl.ds(base, 8)], nrmb.at[0])
    plsc.subcore_barrier()
    pltpu.async_copy(h_h.at[rowb.at[0].at[0]], rows_a, sem_a)
    pltpu.async_copy(h_h.at[rowb.at[0].at[1]], rows_b, sem_b)

    lane = lax.iota(jnp.int32, L)
    lane2 = lane * 2

    def scale(rows_v, par, j):
        j16 = jnp.full((L,), j, jnp.int32)
        p16 = jnp.full((L,), par, jnp.int32)

        @plsc.parallel_loop(0, K, 1, unroll=4)
        def _(e):
            e16 = jnp.full((L,), e, jnp.int32)
            s16 = plsc.load_gather(nrmb, [p16, j16, e16])
            sbf = plsc.pack(s16, s16, format=plsc.PackFormat.INTERLEAVED)
            for q in range(nv // 2):
                w = plsc.load_gather(rows_v, [e16, lane + L * q])
                hv = plsc.bitcast(w, jnp.bfloat16)
                prod = hv * sbf
                pa, pb = plsc.unpack(prod,
                                     format=plsc.PackFormat.INTERLEAVED)
                plsc.store_scatter(stage, [e16, lane + 2 * L * q], pa)
                plsc.store_scatter(stage, [e16, lane + 2 * L * q + L], pb)

    def halfstep(rows_v, sem, c, par, j):
        pltpu.make_async_copy(h_h.at[pl.ds(0, K)], rows_v, sem).wait()
        scale(rows_v, par, j)
        pltpu.sync_copy(stage, acc_sp.at[colb.at[par].at[j]], add=True)
        c2 = c + 2

        @pl.when(c2 < chunks)
        def _():
            blk2 = c2 // 8
            pltpu.async_copy(
                h_h.at[rowb.at[blk2 % 2].at[c2 - 8 * blk2]], rows_v, sem)

    def step(i, carry):
        c0 = 2 * i
        blk = i // 4
        par = blk % 2
        j0 = c0 - 8 * blk

        @pl.when((j0 == 0) & (blk + 1 < nblk))
        def _():
            src = base + 8 * (blk + 1)
            pp = 1 - par
            pltpu.async_copy(row_h.at[pl.ds(src, 8)], rowb.at[pp], sem_p)
            pltpu.async_copy(col_h.at[pl.ds(src, 8)], colb.at[pp], sem_p)
            pltpu.async_copy(nrm_h.at[pl.ds(src, 8)], nrmb.at[pp], sem_p)

        @pl.when((j0 == 6) & (blk + 1 < nblk))
        def _():
            pp = 1 - par
            pltpu.make_async_copy(row_h.at[pl.ds(0, 8)],
                                  rowb.at[pp], sem_p).wait()
            pltpu.make_async_copy(col_h.at[pl.ds(0, 8)],
                                  colb.at[pp], sem_p).wait()
            pltpu.make_async_copy(nrm_h.at[pl.ds(0, 8)],
                                  nrmb.at[pp], sem_p).wait()

        halfstep(rows_a, sem_a, c0, par, j0)
        halfstep(rows_b, sem_b, c0 + 1, par, j0 + 1)
        return carry
    lax.fori_loop(0, chunks // 2, step, 0)
    plsc.subcore_barrier()

    @pl.when(cid == 0)
    def _():
        pltpu.sync_copy(acc_sp.at[pl.ds(npt * sid, npt)],
                        a0_h.at[pl.ds(npt * sid, npt)])

    @pl.when(cid == 1)
    def _():
        pltpu.sync_copy(acc_sp.at[pl.ds(npt * sid, npt)],
                        a1_h.at[pl.ds(npt * sid, npt)])


def _make_msg_kernel(chunks, npad, d):
    return pl.kernel(
        functools.partial(_msg_body, chunks, npad, d),
        out_type=[jax.ShapeDtypeStruct((npad, d), jnp.float32),
                  jax.ShapeDtypeStruct((npad, d), jnp.float32)],
        mesh=_mesh(),
        compiler_params=pltpu.CompilerParams(needs_layout_passes=False,
                                             use_tc_tiling_on_sc=False),
        scratch_types=[
            pltpu.VMEM((2, 8, K), jnp.int32),
            pltpu.VMEM((2, 8, K), jnp.int32),
            pltpu.VMEM((2, 8, K), jnp.float32),
            pltpu.VMEM((K, d // 2), jnp.int32),
            pltpu.VMEM((K, d // 2), jnp.int32),
            pltpu.VMEM((K, d), jnp.float32),
            pltpu.VMEM_SHARED((npad, d), jnp.float32),
            pltpu.SemaphoreType.DMA,
            pltpu.SemaphoreType.DMA,
            pltpu.SemaphoreType.DMA,
        ],
    )


def _pick_bm(n):
    for bm in (1000, 500, 200, 100, 8):
        if n % bm == 0:
            return bm
    return n


def _tc_matmul(x, w, out_dtype=jnp.float32):
    n, d = x.shape
    bm = _pick_bm(n)

    def body(x_ref, w_ref, o_ref):
        o_ref[...] = jnp.dot(x_ref[...], w_ref[...],
                             preferred_element_type=jnp.float32
                             ).astype(out_dtype)
    return pl.pallas_call(
        body,
        grid=(n // bm,),
        in_specs=[pl.BlockSpec((bm, d), lambda j: (j, 0)),
                  pl.BlockSpec(w.shape, lambda j: (0, 0))],
        out_specs=pl.BlockSpec((bm, w.shape[1]), lambda j: (j, 0)),
        out_shape=jax.ShapeDtypeStruct((n, w.shape[1]), out_dtype),
    )(x, w)


def _tc_relu_matmul(n, a0, a1, b, w, bout=None, out_dtype=jnp.float32):
    d = a0.shape[1]
    bm = _pick_bm(n)
    b2 = b.reshape(1, d)
    args = [a0, a1, b2, w]
    in_specs = [pl.BlockSpec((bm, d), lambda j: (j, 0)),
                pl.BlockSpec((bm, d), lambda j: (j, 0)),
                pl.BlockSpec((1, d), lambda j: (0, 0)),
                pl.BlockSpec(w.shape, lambda j: (0, 0))]
    if bout is not None:
        args.append(bout.reshape(1, w.shape[1]))
        in_specs.append(pl.BlockSpec((1, w.shape[1]), lambda j: (0, 0)))

    def body(a0_ref, a1_ref, b_ref, w_ref, *rest):
        o_ref = rest[-1]
        t = jnp.maximum(a0_ref[...] + a1_ref[...] + b_ref[...], 0.0)
        o = jnp.dot(t, w_ref[...], preferred_element_type=jnp.float32)
        if bout is not None:
            o = o + rest[0][...]
        o_ref[...] = o.astype(out_dtype)
    return pl.pallas_call(
        body,
        grid=(n // bm,),
        in_specs=in_specs,
        out_specs=pl.BlockSpec((bm, w.shape[1]), lambda j: (j, 0)),
        out_shape=jax.ShapeDtypeStruct((n, w.shape[1]), out_dtype),
    )(*args)


def kernel(x, edge_index, edge_attr, W1, b1, W2, b2, Wout, bout):
    n, d = x.shape
    e = edge_index.shape[1]

    loop = jnp.arange(n, dtype=jnp.int32)
    ep_real = e + n
    chunks = 8 * (-(-ep_real // (NW * K * 8)))
    ep = NW * K * chunks
    pad = ep - ep_real
    padi = jnp.arange(pad, dtype=jnp.int32)
    row = jnp.concatenate(
        [edge_index[0].astype(jnp.int32), loop,
         padi % n]).reshape(NW * chunks, K)
    col = jnp.concatenate(
        [edge_index[1].astype(jnp.int32), loop,
         padi % n]).reshape(NW * chunks, K)
    ew = jnp.concatenate(
        [edge_attr.astype(jnp.float32), jnp.ones((n,), jnp.float32),
         jnp.zeros((pad,), jnp.float32)]).reshape(NW * chunks, K)

    npt = -(-n // (NS * K)) * K
    npad = NS * npt

    norm = _make_norm_kernel(chunks, npad)(row, col, ew)
    msg = _make_msg_kernel(chunks, npad, d)

    def as_i32_rows(h_bf16):
        return lax.bitcast_convert_type(
            h_bf16.reshape(n, d // 2, 2), jnp.int32)

    qi = jnp.arange(d, dtype=jnp.int32)
    ki = qi % (2 * L)
    perm = (qi // (2 * L)) * (2 * L) + jnp.where(
        ki < L, ki * 2, (ki - L) * 2 + 1)

    h1 = _tc_matmul(x.astype(jnp.float32), W1, out_dtype=jnp.bfloat16)
    a0, a1 = msg(as_i32_rows(h1), row, col, norm)
    h2 = _tc_relu_matmul(n, a0, a1, b1[perm], W2[perm, :],
                         out_dtype=jnp.bfloat16)
    c0, c1 = msg(as_i32_rows(h2), row, col, norm)
    out = _tc_relu_matmul(n, c0, c1, b2[perm], Wout[perm, :], bout)
    return out


# speedup vs baseline: 1.1111x; 1.0824x over previous
import functools

import jax
import jax.numpy as jnp
from jax import lax
from jax.experimental import pallas as pl
from jax.experimental.pallas import tpu as pltpu
from jax.experimental.pallas import tpu_sc as plsc

NC = 2
NS = 16
NW = NC * NS
L = 16
K = 128


def _mesh():
    return plsc.VectorSubcoreMesh(
        core_axis_name="c", subcore_axis_name="s", num_cores=NC,
        num_subcores=NS)


def _norm_body(chunks, npad, row_h, col_h, ew_h, norm_h,
               row_v, col_v, ew_v, dis_v, nrm_v, t_v, acc_sp, sem_s):
    cid = lax.axis_index("c")
    sid = lax.axis_index("s")
    npt = npad // NS
    cpt = 2 * chunks

    @pl.when(cid == 0)
    def _():
        zero = jnp.zeros((L,), jnp.float32)
        pltpu.sync_copy(col_h.at[pl.ds(cpt * sid, cpt)], col_v)
        pltpu.sync_copy(ew_h.at[pl.ds(cpt * sid, cpt)], ew_v)
        for i in range(npt // L):
            t_v[pl.ds(L * i, L)] = zero
        pltpu.sync_copy(t_v, acc_sp.at[pl.ds(npt * sid, npt)])
        plsc.subcore_barrier()

        def step_a(ci, carry):
            pltpu.async_copy(ew_v.at[ci], acc_sp.at[col_v.at[ci]], sem_s,
                             add=True)
            return carry
        lax.fori_loop(0, cpt, step_a, 0)

        def drain_a(ci, carry):
            pltpu.make_async_copy(ew_v.at[0], acc_sp.at[col_v.at[0]],
                                  sem_s).wait()
            return carry
        lax.fori_loop(0, cpt, drain_a, 0)
        plsc.subcore_barrier()

        pltpu.sync_copy(acc_sp.at[pl.ds(npt * sid, npt)], t_v)
        for i in range(npt // L):
            d = t_v[pl.ds(L * i, L)]
            yi = lax.bitcast_convert_type(d, jnp.int32)
            yi = jnp.int32(0x5F3759DF) - lax.shift_right_logical(yi, 1)
            y = lax.bitcast_convert_type(yi, jnp.float32)
            for _ in range(3):
                y = y * (1.5 - 0.5 * d * y * y)
            t_v[pl.ds(L * i, L)] = y
        pltpu.sync_copy(t_v, acc_sp.at[pl.ds(npt * sid, npt)])
        plsc.subcore_barrier()
        pltpu.sync_copy(acc_sp, dis_v)

        pltpu.sync_copy(row_h.at[pl.ds(cpt * sid, cpt)], row_v)
        lane = lax.iota(jnp.int32, L)

        @plsc.parallel_loop(0, cpt * (K // L), 1, unroll=4)
        def _(g):
            ci16 = jnp.full((L,), g // (K // L), jnp.int32)
            off = lane + L * (g % (K // L))
            r16 = plsc.load_gather(row_v, [ci16, off])
            c16 = plsc.load_gather(col_v, [ci16, off])
            w16 = plsc.load_gather(ew_v, [ci16, off])
            dr = plsc.load_gather(dis_v, [r16])
            dc = plsc.load_gather(dis_v, [c16])
            plsc.store_scatter(nrm_v, [ci16, off], dr * w16 * dc)
        pltpu.sync_copy(nrm_v, norm_h.at[pl.ds(cpt * sid, cpt)])


def _make_norm_kernel(chunks, npad):
    cpt = 2 * chunks
    return pl.kernel(
        functools.partial(_norm_body, chunks, npad),
        out_type=jax.ShapeDtypeStruct((NW * chunks, K), jnp.float32),
        mesh=_mesh(),
        compiler_params=pltpu.CompilerParams(needs_layout_passes=False),
        scratch_types=[
            pltpu.VMEM((cpt, K), jnp.int32),
            pltpu.VMEM((cpt, K), jnp.int32),
            pltpu.VMEM((cpt, K), jnp.float32),
            pltpu.VMEM((npad,), jnp.float32),
            pltpu.VMEM((cpt, K), jnp.float32),
            pltpu.VMEM((npad // NS,), jnp.float32),
            pltpu.VMEM_SHARED((npad,), jnp.float32),
            pltpu.SemaphoreType.DMA,
        ],
    )


def _msg_body(chunks, npad, d,
              h_h, row_h, col_h, nrm_h, a0_h, a1_h,
              rowb, colb, nrmb, rows_a, rows_b, acc_sp,
              sem_a, sem_b, sem_p, sem_sa, sem_sb):
    cid = lax.axis_index("c")
    sid = lax.axis_index("s")
    wid = 2 * sid + cid
    npt = npad // NS
    nv = d // L
    base = chunks * wid
    nblk = chunks // 8

    zero = jnp.zeros((L,), jnp.float32)

    def zr(r, carry):
        for v in range(nv):
            rows_a[r, pl.ds(L * v, L)] = zero
        return carry
    lax.fori_loop(0, K, zr, 0)
    for k in range(npt // K):
        pltpu.sync_copy(rows_a, acc_sp.at[pl.ds(npt * sid + K * k, K)])
    pltpu.sync_copy(row_h.at[pl.ds(base, 8)], rowb.at[0])
    pltpu.sync_copy(col_h.at[pl.ds(base, 8)], colb.at[0])
    pltpu.sync_copy(nrm_h.at[pl.ds(base, 8)], nrmb.at[0])
    plsc.subcore_barrier()
    pltpu.async_copy(h_h.at[rowb.at[0].at[0]], rows_a, sem_a)
    pltpu.async_copy(h_h.at[rowb.at[0].at[1]], rows_b, sem_b)

    lane = lax.iota(jnp.int32, L)

    def scale(rows_v, par, j):
        j16 = jnp.full((L,), j, jnp.int32)
        p16 = jnp.full((L,), par, jnp.int32)

        @plsc.parallel_loop(0, K, 1, unroll=4)
        def _(e):
            e16 = jnp.full((L,), e, jnp.int32)
            s16 = plsc.load_gather(nrmb, [p16, j16, e16])
            for v in range(nv):
                off = lane + L * v
                x = plsc.load_gather(rows_v, [e16, off])
                plsc.store_scatter(rows_v, [e16, off], x * s16)
    def halfstep(rows_v, sem, c, par, j):
        pltpu.make_async_copy(h_h.at[pl.ds(0, K)], rows_v, sem).wait()
        scale(rows_v, par, j)
        pltpu.sync_copy(rows_v, acc_sp.at[colb.at[par].at[j]], add=True)
        c2 = c + 2

        @pl.when(c2 < chunks)
        def _():
            blk2 = c2 // 8
            pltpu.async_copy(
                h_h.at[rowb.at[blk2 % 2].at[c2 - 8 * blk2]], rows_v, sem)

    def step(i, carry):
        c0 = 2 * i
        blk = i // 4
        par = blk % 2
        j0 = c0 - 8 * blk

        @pl.when((j0 == 0) & (blk + 1 < nblk))
        def _():
            src = base + 8 * (blk + 1)
            pp = 1 - par
            pltpu.async_copy(row_h.at[pl.ds(src, 8)], rowb.at[pp], sem_p)
            pltpu.async_copy(col_h.at[pl.ds(src, 8)], colb.at[pp], sem_p)
            pltpu.async_copy(nrm_h.at[pl.ds(src, 8)], nrmb.at[pp], sem_p)

        @pl.when((j0 == 6) & (blk + 1 < nblk))
        def _():
            pp = 1 - par
            pltpu.make_async_copy(row_h.at[pl.ds(0, 8)],
                                  rowb.at[pp], sem_p).wait()
            pltpu.make_async_copy(col_h.at[pl.ds(0, 8)],
                                  colb.at[pp], sem_p).wait()
            pltpu.make_async_copy(nrm_h.at[pl.ds(0, 8)],
                                  nrmb.at[pp], sem_p).wait()

        halfstep(rows_a, sem_a, c0, par, j0)
        halfstep(rows_b, sem_b, c0 + 1, par, j0 + 1)
        return carry
    lax.fori_loop(0, chunks // 2, step, 0)
    plsc.subcore_barrier()

    @pl.when(cid == 0)
    def _():
        pltpu.sync_copy(acc_sp.at[pl.ds(npt * sid, npt)],
                        a0_h.at[pl.ds(npt * sid, npt)])

    @pl.when(cid == 1)
    def _():
        pltpu.sync_copy(acc_sp.at[pl.ds(npt * sid, npt)],
                        a1_h.at[pl.ds(npt * sid, npt)])


def _make_msg_kernel(chunks, npad, d):
    return pl.kernel(
        functools.partial(_msg_body, chunks, npad, d),
        out_type=[jax.ShapeDtypeStruct((npad, d), jnp.float32),
                  jax.ShapeDtypeStruct((npad, d), jnp.float32)],
        mesh=_mesh(),
        compiler_params=pltpu.CompilerParams(needs_layout_passes=False),
        scratch_types=[
            pltpu.VMEM((2, 8, K), jnp.int32),
            pltpu.VMEM((2, 8, K), jnp.int32),
            pltpu.VMEM((2, 8, K), jnp.float32),
            pltpu.VMEM((K, d), jnp.float32),
            pltpu.VMEM((K, d), jnp.float32),
            pltpu.VMEM_SHARED((npad, d), jnp.float32),
            pltpu.SemaphoreType.DMA,
            pltpu.SemaphoreType.DMA,
            pltpu.SemaphoreType.DMA,
            pltpu.SemaphoreType.DMA,
            pltpu.SemaphoreType.DMA,
        ],
    )


def _pick_bm(n):
    for bm in (1000, 500, 200, 100, 8):
        if n % bm == 0:
            return bm
    return n


def _tc_matmul(x, w):
    n, d = x.shape
    bm = _pick_bm(n)

    def body(x_ref, w_ref, o_ref):
        o_ref[...] = jnp.dot(x_ref[...], w_ref[...],
                             preferred_element_type=jnp.float32)
    return pl.pallas_call(
        body,
        grid=(n // bm,),
        in_specs=[pl.BlockSpec((bm, d), lambda j: (j, 0)),
                  pl.BlockSpec(w.shape, lambda j: (0, 0))],
        out_specs=pl.BlockSpec((bm, w.shape[1]), lambda j: (j, 0)),
        out_shape=jax.ShapeDtypeStruct((n, w.shape[1]), jnp.float32),
    )(x, w)


def _tc_relu_matmul(n, a0, a1, b, w, bout=None):
    d = a0.shape[1]
    bm = _pick_bm(n)
    b2 = b.reshape(1, d)
    args = [a0, a1, b2, w]
    in_specs = [pl.BlockSpec((bm, d), lambda j: (j, 0)),
                pl.BlockSpec((bm, d), lambda j: (j, 0)),
                pl.BlockSpec((1, d), lambda j: (0, 0)),
                pl.BlockSpec(w.shape, lambda j: (0, 0))]
    if bout is not None:
        args.append(bout.reshape(1, w.shape[1]))
        in_specs.append(pl.BlockSpec((1, w.shape[1]), lambda j: (0, 0)))

    def body(a0_ref, a1_ref, b_ref, w_ref, *rest):
        o_ref = rest[-1]
        t = jnp.maximum(a0_ref[...] + a1_ref[...] + b_ref[...], 0.0)
        o = jnp.dot(t, w_ref[...], preferred_element_type=jnp.float32)
        if bout is not None:
            o = o + rest[0][...]
        o_ref[...] = o
    return pl.pallas_call(
        body,
        grid=(n // bm,),
        in_specs=in_specs,
        out_specs=pl.BlockSpec((bm, w.shape[1]), lambda j: (j, 0)),
        out_shape=jax.ShapeDtypeStruct((n, w.shape[1]), jnp.float32),
    )(*args)


def kernel(x, edge_index, edge_attr, W1, b1, W2, b2, Wout, bout):
    n, d = x.shape
    e = edge_index.shape[1]

    loop = jnp.arange(n, dtype=jnp.int32)
    ep_real = e + n
    chunks = 8 * (-(-ep_real // (NW * K * 8)))
    ep = NW * K * chunks
    pad = ep - ep_real
    padi = jnp.arange(pad, dtype=jnp.int32)
    row = jnp.concatenate(
        [edge_index[0].astype(jnp.int32), loop,
         padi % n]).reshape(NW * chunks, K)
    col = jnp.concatenate(
        [edge_index[1].astype(jnp.int32), loop,
         padi % n]).reshape(NW * chunks, K)
    ew = jnp.concatenate(
        [edge_attr.astype(jnp.float32), jnp.ones((n,), jnp.float32),
         jnp.zeros((pad,), jnp.float32)]).reshape(NW * chunks, K)

    npt = -(-n // (NS * K)) * K
    npad = NS * npt

    norm = _make_norm_kernel(chunks, npad)(row, col, ew)
    msg = _make_msg_kernel(chunks, npad, d)

    h1 = _tc_matmul(x.astype(jnp.float32), W1)
    a0, a1 = msg(h1, row, col, norm)
    h2 = _tc_relu_matmul(n, a0, a1, b1, W2)
    c0, c1 = msg(h2, row, col, norm)
    out = _tc_relu_matmul(n, c0, c1, b2, Wout, bout)
    return out


# speedup vs baseline: 1.1269x; 1.0142x over previous
import functools

import jax
import jax.numpy as jnp
from jax import lax
from jax.experimental import pallas as pl
from jax.experimental.pallas import tpu as pltpu
from jax.experimental.pallas import tpu_sc as plsc

NC = 2
NS = 16
NW = NC * NS
L = 16
K = 128


def _mesh():
    return plsc.VectorSubcoreMesh(
        core_axis_name="c", subcore_axis_name="s", num_cores=NC,
        num_subcores=NS)


def _norm_body(chunks, npad, row_h, col_h, ew_h, norm_h,
               row_v, col_v, ew_v, dis_v, nrm_v, t_v, acc_sp, sem_s):
    cid = lax.axis_index("c")
    sid = lax.axis_index("s")
    npt = npad // NS
    cpt = 2 * chunks

    @pl.when(cid == 0)
    def _():
        zero = jnp.zeros((L,), jnp.float32)
        pltpu.sync_copy(col_h.at[pl.ds(cpt * sid, cpt)], col_v)
        pltpu.sync_copy(ew_h.at[pl.ds(cpt * sid, cpt)], ew_v)
        for i in range(npt // L):
            t_v[pl.ds(L * i, L)] = zero
        pltpu.sync_copy(t_v, acc_sp.at[pl.ds(npt * sid, npt)])
        plsc.subcore_barrier()

        def step_a(ci, carry):
            pltpu.async_copy(ew_v.at[ci], acc_sp.at[col_v.at[ci]], sem_s,
                             add=True)
            return carry
        lax.fori_loop(0, cpt, step_a, 0)

        def drain_a(ci, carry):
            pltpu.make_async_copy(ew_v.at[0], acc_sp.at[col_v.at[0]],
                                  sem_s).wait()
            return carry
        lax.fori_loop(0, cpt, drain_a, 0)
        plsc.subcore_barrier()

        pltpu.sync_copy(acc_sp.at[pl.ds(npt * sid, npt)], t_v)
        for i in range(npt // L):
            d = t_v[pl.ds(L * i, L)]
            yi = lax.bitcast_convert_type(d, jnp.int32)
            yi = jnp.int32(0x5F3759DF) - lax.shift_right_logical(yi, 1)
            y = lax.bitcast_convert_type(yi, jnp.float32)
            for _ in range(3):
                y = y * (1.5 - 0.5 * d * y * y)
            t_v[pl.ds(L * i, L)] = y
        pltpu.sync_copy(t_v, acc_sp.at[pl.ds(npt * sid, npt)])
        plsc.subcore_barrier()
        pltpu.sync_copy(acc_sp, dis_v)

        pltpu.sync_copy(row_h.at[pl.ds(cpt * sid, cpt)], row_v)
        lane = lax.iota(jnp.int32, L)

        @plsc.parallel_loop(0, cpt * (K // L), 1, unroll=4)
        def _(g):
            ci16 = jnp.full((L,), g // (K // L), jnp.int32)
            off = lane + L * (g % (K // L))
            r16 = plsc.load_gather(row_v, [ci16, off])
            c16 = plsc.load_gather(col_v, [ci16, off])
            w16 = plsc.load_gather(ew_v, [ci16, off])
            dr = plsc.load_gather(dis_v, [r16])
            dc = plsc.load_gather(dis_v, [c16])
            plsc.store_scatter(nrm_v, [ci16, off], dr * w16 * dc)
        pltpu.sync_copy(nrm_v, norm_h.at[pl.ds(cpt * sid, cpt)])


def _make_norm_kernel(chunks, npad):
    cpt = 2 * chunks
    return pl.kernel(
        functools.partial(_norm_body, chunks, npad),
        out_type=jax.ShapeDtypeStruct((NW * chunks, K), jnp.float32),
        mesh=_mesh(),
        compiler_params=pltpu.CompilerParams(needs_layout_passes=False),
        scratch_types=[
            pltpu.VMEM((cpt, K), jnp.int32),
            pltpu.VMEM((cpt, K), jnp.int32),
            pltpu.VMEM((cpt, K), jnp.float32),
            pltpu.VMEM((npad,), jnp.float32),
            pltpu.VMEM((cpt, K), jnp.float32),
            pltpu.VMEM((npad // NS,), jnp.float32),
            pltpu.VMEM_SHARED((npad,), jnp.float32),
            pltpu.SemaphoreType.DMA,
        ],
    )


def _msg_body(chunks, npad, d,
              h_h, row_h, col_h, nrm_h, a0_h, a1_h,
              rowb, colb, nrmb, rows_a, rows_b, acc_sp,
              sem_a, sem_b, sem_p):
    cid = lax.axis_index("c")
    sid = lax.axis_index("s")
    wid = 2 * sid + cid
    npt = npad // NS
    nv = d // L
    base = chunks * wid
    nblk = chunks // 8

    zero = jnp.zeros((L,), jnp.float32)

    def zr(r, carry):
        for v in range(nv):
            rows_a[r, pl.ds(L * v, L)] = zero
        return carry
    lax.fori_loop(0, K, zr, 0)
    for k in range(npt // K):
        pltpu.sync_copy(rows_a, acc_sp.at[pl.ds(npt * sid + K * k, K)])
    pltpu.sync_copy(row_h.at[pl.ds(base, 8)], rowb.at[0])
    pltpu.sync_copy(col_h.at[pl.ds(base, 8)], colb.at[0])
    pltpu.sync_copy(nrm_h.at[pl.ds(base, 8)], nrmb.at[0])
    plsc.subcore_barrier()
    pltpu.async_copy(h_h.at[rowb.at[0].at[0]], rows_a, sem_a)
    pltpu.async_copy(h_h.at[rowb.at[0].at[1]], rows_b, sem_b)

    lane = lax.iota(jnp.int32, L)

    def scale(rows_v, par, j):
        j16 = jnp.full((L,), j, jnp.int32)
        p16 = jnp.full((L,), par, jnp.int32)

        @plsc.parallel_loop(0, K, 1, unroll=4)
        def _(e):
            e16 = jnp.full((L,), e, jnp.int32)
            s16 = plsc.load_gather(nrmb, [p16, j16, e16])
            for v in range(nv):
                off = lane + L * v
                x = plsc.load_gather(rows_v, [e16, off])
                plsc.store_scatter(rows_v, [e16, off], x * s16)
    def halfstep(rows_v, sem, c, par, j):
        pltpu.make_async_copy(h_h.at[pl.ds(0, K)], rows_v, sem).wait()
        scale(rows_v, par, j)
        pltpu.sync_copy(rows_v, acc_sp.at[colb.at[par].at[j]], add=True)
        c2 = c + 2

        @pl.when(c2 < chunks)
        def _():
            blk2 = c2 // 8
            pltpu.async_copy(
                h_h.at[rowb.at[blk2 % 2].at[c2 - 8 * blk2]], rows_v, sem)

    def step(i, carry):
        c0 = 2 * i
        blk = i // 4
        par = blk % 2
        j0 = c0 - 8 * blk

        @pl.when((j0 == 0) & (blk + 1 < nblk))
        def _():
            src = base + 8 * (blk + 1)
            pp = 1 - par
            pltpu.async_copy(row_h.at[pl.ds(src, 8)], rowb.at[pp], sem_p)
            pltpu.async_copy(col_h.at[pl.ds(src, 8)], colb.at[pp], sem_p)
            pltpu.async_copy(nrm_h.at[pl.ds(src, 8)], nrmb.at[pp], sem_p)

        @pl.when((j0 == 6) & (blk + 1 < nblk))
        def _():
            pp = 1 - par
            pltpu.make_async_copy(row_h.at[pl.ds(0, 8)],
                                  rowb.at[pp], sem_p).wait()
            pltpu.make_async_copy(col_h.at[pl.ds(0, 8)],
                                  colb.at[pp], sem_p).wait()
            pltpu.make_async_copy(nrm_h.at[pl.ds(0, 8)],
                                  nrmb.at[pp], sem_p).wait()

        halfstep(rows_a, sem_a, c0, par, j0)
        halfstep(rows_b, sem_b, c0 + 1, par, j0 + 1)
        return carry
    lax.fori_loop(0, chunks // 2, step, 0)
    plsc.subcore_barrier()

    @pl.when(cid == 0)
    def _():
        pltpu.sync_copy(acc_sp.at[pl.ds(npt * sid, npt)],
                        a0_h.at[pl.ds(npt * sid, npt)])

    @pl.when(cid == 1)
    def _():
        pltpu.sync_copy(acc_sp.at[pl.ds(npt * sid, npt)],
                        a1_h.at[pl.ds(npt * sid, npt)])


def _make_msg_kernel(chunks, npad, d):
    return pl.kernel(
        functools.partial(_msg_body, chunks, npad, d),
        out_type=[jax.ShapeDtypeStruct((npad, d), jnp.float32),
                  jax.ShapeDtypeStruct((npad, d), jnp.float32)],
        mesh=_mesh(),
        compiler_params=pltpu.CompilerParams(needs_layout_passes=False),
        scratch_types=[
            pltpu.VMEM((2, 8, K), jnp.int32),
            pltpu.VMEM((2, 8, K), jnp.int32),
            pltpu.VMEM((2, 8, K), jnp.float32),
            pltpu.VMEM((K, d), jnp.float32),
            pltpu.VMEM((K, d), jnp.float32),
            pltpu.VMEM_SHARED((npad, d), jnp.float32),
            pltpu.SemaphoreType.DMA,
            pltpu.SemaphoreType.DMA,
            pltpu.SemaphoreType.DMA,
        ],
    )


def _pick_bm(n):
    for bm in (2000, 1000, 500, 200, 100, 8):
        if n % bm == 0:
            return bm
    return n


def _tc_matmul(x, w):
    n, d = x.shape
    bm = _pick_bm(n)

    def body(x_ref, w_ref, o_ref):
        o_ref[...] = jnp.dot(x_ref[...], w_ref[...],
                             preferred_element_type=jnp.float32)
    return pl.pallas_call(
        body,
        grid=(n // bm,),
        in_specs=[pl.BlockSpec((bm, d), lambda j: (j, 0)),
                  pl.BlockSpec(w.shape, lambda j: (0, 0))],
        out_specs=pl.BlockSpec((bm, w.shape[1]), lambda j: (j, 0)),
        out_shape=jax.ShapeDtypeStruct((n, w.shape[1]), jnp.float32),
    )(x, w)


def _tc_relu_matmul(n, a0, a1, b, w, bout=None):
    d = a0.shape[1]
    bm = _pick_bm(n)
    b2 = b.reshape(1, d)
    args = [a0, a1, b2, w]
    in_specs = [pl.BlockSpec((bm, d), lambda j: (j, 0)),
                pl.BlockSpec((bm, d), lambda j: (j, 0)),
                pl.BlockSpec((1, d), lambda j: (0, 0)),
                pl.BlockSpec(w.shape, lambda j: (0, 0))]
    if bout is not None:
        args.append(bout.reshape(1, w.shape[1]))
        in_specs.append(pl.BlockSpec((1, w.shape[1]), lambda j: (0, 0)))

    def body(a0_ref, a1_ref, b_ref, w_ref, *rest):
        o_ref = rest[-1]
        t = jnp.maximum(a0_ref[...] + a1_ref[...] + b_ref[...], 0.0)
        o = jnp.dot(t, w_ref[...], preferred_element_type=jnp.float32)
        if bout is not None:
            o = o + rest[0][...]
        o_ref[...] = o
    return pl.pallas_call(
        body,
        grid=(n // bm,),
        in_specs=in_specs,
        out_specs=pl.BlockSpec((bm, w.shape[1]), lambda j: (j, 0)),
        out_shape=jax.ShapeDtypeStruct((n, w.shape[1]), jnp.float32),
    )(*args)


def kernel(x, edge_index, edge_attr, W1, b1, W2, b2, Wout, bout):
    n, d = x.shape
    e = edge_index.shape[1]

    loop = jnp.arange(n, dtype=jnp.int32)
    ep_real = e + n
    chunks = 8 * (-(-ep_real // (NW * K * 8)))
    ep = NW * K * chunks
    pad = ep - ep_real
    padi = jnp.arange(pad, dtype=jnp.int32)
    row = jnp.concatenate(
        [edge_index[0].astype(jnp.int32), loop,
         padi % n]).reshape(NW * chunks, K)
    col = jnp.concatenate(
        [edge_index[1].astype(jnp.int32), loop,
         padi % n]).reshape(NW * chunks, K)
    ew = jnp.concatenate(
        [edge_attr.astype(jnp.float32), jnp.ones((n,), jnp.float32),
         jnp.zeros((pad,), jnp.float32)]).reshape(NW * chunks, K)

    npt = -(-n // (NS * K)) * K
    npad = NS * npt

    norm = _make_norm_kernel(chunks, npad)(row, col, ew)
    msg = _make_msg_kernel(chunks, npad, d)

    h1 = _tc_matmul(x.astype(jnp.float32), W1)
    a0, a1 = msg(h1, row, col, norm)
    h2 = _tc_relu_matmul(n, a0, a1, b1, W2)
    c0, c1 = msg(h2, row, col, norm)
    out = _tc_relu_matmul(n, c0, c1, b2, Wout, bout)
    return out
